# baseline (speedup 1.0000x reference)
import jax
import jax.numpy as jnp
from jax.experimental import pallas as pl
from jax.experimental.pallas import tpu as pltpu

_BLOCKN = 32768


def _make_kernel(n, nn, edge_f, edge_out, in_f, out_f, nsteps, rows):
    def fused(ftb_ref, wqt_ref, wrt_ref, x_ref, w_ref, b_ref,
              fout_ref, out_ref, a_scr, sup_scr):
        i = pl.program_id(0)

        @pl.when(i == 0)
        def _():
            sup_scr[...] = jnp.dot(x_ref[...], w_ref[...],
                                   preferred_element_type=jnp.float32)

        ftb = ftb_ref[...]
        fout_ref[...] = jnp.dot(wrt_ref[...], ftb,
                                preferred_element_type=jnp.float32)
        arow = jnp.dot(wqt_ref[...], ftb,
                       preferred_element_type=jnp.float32)
        a_scr[pl.ds(i * rows, rows), :] = arow.reshape(rows, n)

        @pl.when(i == nsteps - 1)
        def _():
            out_ref[...] = (
                jnp.dot(a_scr[...], sup_scr[...],
                        preferred_element_type=jnp.float32)
                + b_ref[...]
            )

    return fused


def kernel(input, adj, F, weight, weight_q, weight_r, bias):
    n, in_f = input.shape
    out_f = weight.shape[1]
    nn, edge_f = F.shape
    edge_out = weight_r.shape[1]

    ft = F.T
    wqt = weight_q.T
    wrt = weight_r.T

    nsteps = nn // _BLOCKN
    rows = _BLOCKN // n

    fused = _make_kernel(n, nn, edge_f, edge_out, in_f, out_f, nsteps, rows)

    fout_t, output = pl.pallas_call(
        fused,
        grid=(nsteps,),
        in_specs=[
            pl.BlockSpec((edge_f, _BLOCKN), lambda i: (0, i)),
            pl.BlockSpec((1, edge_f), lambda i: (0, 0)),
            pl.BlockSpec((edge_out, edge_f), lambda i: (0, 0)),
            pl.BlockSpec((n, in_f), lambda i: (0, 0)),
            pl.BlockSpec((in_f, out_f), lambda i: (0, 0)),
            pl.BlockSpec((1, out_f), lambda i: (0, 0)),
        ],
        out_specs=[
            pl.BlockSpec((edge_out, _BLOCKN), lambda i: (0, i)),
            pl.BlockSpec((n, out_f), lambda i: (0, 0)),
        ],
        out_shape=[
            jax.ShapeDtypeStruct((edge_out, nn), jnp.float32),
            jax.ShapeDtypeStruct((n, out_f), jnp.float32),
        ],
        scratch_shapes=[
            pltpu.VMEM((n, n), jnp.float32),
            pltpu.VMEM((n, out_f), jnp.float32),
        ],
    )(ft, wqt, wrt, input, weight, bias.reshape(1, out_f))

    return (output, fout_t.T)


# speedup vs baseline: 1.1820x; 1.1820x over previous
import jax
import jax.numpy as jnp
from jax.experimental import pallas as pl
from jax.experimental.pallas import tpu as pltpu

_BLOCKN = 131072


def _make_kernel(n, nn, edge_f, edge_out, in_f, out_f, nsteps, rows):
    def fused(ftb_ref, wqt_ref, wrt_ref, x_ref, w_ref, b_ref,
              fout_ref, out_ref, a_scr, sup_scr):
        i = pl.program_id(0)

        @pl.when(i == 0)
        def _():
            sup_scr[...] = jnp.dot(x_ref[...], w_ref[...],
                                   preferred_element_type=jnp.float32)

        ftb = ftb_ref[...]
        fout_ref[...] = jnp.dot(wrt_ref[...], ftb,
                                preferred_element_type=jnp.float32)
        arow = jnp.dot(wqt_ref[...], ftb,
                       preferred_element_type=jnp.float32)
        a_scr[pl.ds(i * rows, rows), :] = arow.reshape(rows, n)

        @pl.when(i == nsteps - 1)
        def _():
            out_ref[...] = (
                jnp.dot(a_scr[...], sup_scr[...],
                        preferred_element_type=jnp.float32)
                + b_ref[...]
            )

    return fused


def kernel(input, adj, F, weight, weight_q, weight_r, bias):
    n, in_f = input.shape
    out_f = weight.shape[1]
    nn, edge_f = F.shape
    edge_out = weight_r.shape[1]

    ft = F.T
    wqt = weight_q.T
    wrt = weight_r.T

    nsteps = nn // _BLOCKN
    rows = _BLOCKN // n

    fused = _make_kernel(n, nn, edge_f, edge_out, in_f, out_f, nsteps, rows)

    fout_t, output = pl.pallas_call(
        fused,
        grid=(nsteps,),
        in_specs=[
            pl.BlockSpec((edge_f, _BLOCKN), lambda i: (0, i)),
            pl.BlockSpec((1, edge_f), lambda i: (0, 0)),
            pl.BlockSpec((edge_out, edge_f), lambda i: (0, 0)),
            pl.BlockSpec((n, in_f), lambda i: (0, 0)),
            pl.BlockSpec((in_f, out_f), lambda i: (0, 0)),
            pl.BlockSpec((1, out_f), lambda i: (0, 0)),
        ],
        out_specs=[
            pl.BlockSpec((edge_out, _BLOCKN), lambda i: (0, i)),
            pl.BlockSpec((n, out_f), lambda i: (0, 0)),
        ],
        out_shape=[
            jax.ShapeDtypeStruct((edge_out, nn), jnp.float32),
            jax.ShapeDtypeStruct((n, out_f), jnp.float32),
        ],
        scratch_shapes=[
            pltpu.VMEM((n, n), jnp.float32),
            pltpu.VMEM((n, out_f), jnp.float32),
        ],
    )(ft, wqt, wrt, input, weight, bias.reshape(1, out_f))

    return (output, fout_t.T)


# speedup vs baseline: 1.2300x; 1.0406x over previous
import jax
import jax.numpy as jnp
from jax.experimental import pallas as pl
from jax.experimental.pallas import tpu as pltpu

_BLOCKN = 131072


def _make_kernel(n, nn, edge_f, edge_out, in_f, out_f, nsteps, rows):
    def fused(ftb_ref, wqt_ref, wrt_ref, x_ref, w_ref, b_ref,
              fout_ref, out_ref, sup_scr):
        i = pl.program_id(0)

        @pl.when(i == 0)
        def _():
            sup_scr[...] = jnp.dot(x_ref[...], w_ref[...],
                                   preferred_element_type=jnp.float32)

        ftb = ftb_ref[...]
        fout_ref[...] = jnp.dot(wrt_ref[...], ftb,
                                preferred_element_type=jnp.float32)
        arow = jnp.dot(wqt_ref[...], ftb,
                       preferred_element_type=jnp.float32)
        out_ref[...] = (
            jnp.dot(arow.reshape(rows, n), sup_scr[...],
                    preferred_element_type=jnp.float32)
            + b_ref[...]
        )

    return fused


def kernel(input, adj, F, weight, weight_q, weight_r, bias):
    n, in_f = input.shape
    out_f = weight.shape[1]
    nn, edge_f = F.shape
    edge_out = weight_r.shape[1]

    ft = F.T
    wqt = weight_q.T
    wrt = weight_r.T

    nsteps = nn // _BLOCKN
    rows = _BLOCKN // n

    fused = _make_kernel(n, nn, edge_f, edge_out, in_f, out_f, nsteps, rows)

    fout_t, output = pl.pallas_call(
        fused,
        grid=(nsteps,),
        in_specs=[
            pl.BlockSpec((edge_f, _BLOCKN), lambda i: (0, i)),
            pl.BlockSpec((1, edge_f), lambda i: (0, 0)),
            pl.BlockSpec((edge_out, edge_f), lambda i: (0, 0)),
            pl.BlockSpec((n, in_f), lambda i: (0, 0)),
            pl.BlockSpec((in_f, out_f), lambda i: (0, 0)),
            pl.BlockSpec((1, out_f), lambda i: (0, 0)),
        ],
        out_specs=[
            pl.BlockSpec((edge_out, _BLOCKN), lambda i: (0, i)),
            pl.BlockSpec((rows, out_f), lambda i: (i, 0)),
        ],
        out_shape=[
            jax.ShapeDtypeStruct((edge_out, nn), jnp.float32),
            jax.ShapeDtypeStruct((n, out_f), jnp.float32),
        ],
        scratch_shapes=[
            pltpu.VMEM((n, out_f), jnp.float32),
        ],
    )(ft, wqt, wrt, input, weight, bias.reshape(1, out_f))

    return (output, fout_t.T)
